# baseline (speedup 1.0000x reference)
import math

import jax
import jax.numpy as jnp
from jax import lax
from jax.experimental import pallas as pl
from jax.experimental.pallas import tpu as pltpu
from jax.experimental.pallas import tpu_sc as plsc

B, L = 1024, 50
ROWS = B * L
IN_W = 41
OUT_W = 395
NC, NS, LANES = 2, 16, 16
NW = NC * NS
ROWS_PER_TILE = ROWS // NW
CHUNK = 80
NCHUNK = ROWS_PER_TILE // CHUNK
NPAIR = NCHUNK // 2
GROUPS = CHUNK // LANES

_TWO_PI = 2.0 * math.pi

TSTRIDE = 41


def _sincos_2pi(x):
    half = jnp.where(x >= 0.0, jnp.float32(0.5), jnp.float32(-0.5))
    k = (x + half).astype(jnp.int32)
    f = x - k.astype(jnp.float32)
    f4 = 4.0 * f
    half2 = jnp.where(f4 >= 0.0, jnp.float32(0.5), jnp.float32(-0.5))
    q = (f4 + half2).astype(jnp.int32)
    v = f - 0.25 * q.astype(jnp.float32)
    t = jnp.float32(_TWO_PI) * v
    t2 = t * t
    s = t * (1.0 + t2 * (-1.0 / 6.0 + t2 * (1.0 / 120.0 + t2 * (-1.0 / 5040.0))))
    c = 1.0 + t2 * (-0.5 + t2 * (1.0 / 24.0 + t2 * (-1.0 / 720.0 + t2 * (1.0 / 40320.0))))
    qm = jnp.bitwise_and(q, 3)
    sin_r = jnp.where(qm == 0, s, jnp.where(qm == 1, c, jnp.where(qm == 2, -s, -c)))
    cos_r = jnp.where(qm == 0, c, jnp.where(qm == 1, -s, jnp.where(qm == 2, -c, s)))
    return sin_r, cos_r


def _body(in_hbm, atom_hbm, num_hbm, out_hbm,
          atom_v, num_v, in_v, out_v0, out_v1, sem0, sem1):
    wid = lax.axis_index("s") * NC + lax.axis_index("c")
    tile_base = wid * ROWS_PER_TILE

    pltpu.sync_copy(atom_hbm, atom_v)
    pltpu.sync_copy(num_hbm, num_v)

    def compute(ci, out_v):
        base = tile_base + ci * CHUNK
        pltpu.sync_copy(in_hbm.at[pl.ds(base * IN_W, CHUNK * IN_W)], in_v)

        @pl.loop(0, GROUPS)
        def _group(g):
            rows = lax.iota(jnp.int32, LANES) + g * LANES
            r_in = rows * IN_W
            r_out = rows * OUT_W

            BATCH = 16

            def move(src, src_base, dst_base, width):
                for c0 in range(0, width, BATCH):
                    n = min(BATCH, width - c0)
                    vals = [plsc.load_gather(src, [src_base + (c0 + k)])
                            for k in range(n)]
                    for k in range(n):
                        plsc.store_scatter(out_v, [dst_base + (c0 + k)], vals[k])

            ia = plsc.load_gather(in_v, [r_in]).astype(jnp.int32) * TSTRIDE
            move(atom_v, ia, r_out, 40)

            for a in range(3):
                x = plsc.load_gather(in_v, [r_in + (1 + a)])
                s, c = _sincos_2pi(x)
                plsc.store_scatter(out_v, [r_out + (40 + 2 * a)], s)
                plsc.store_scatter(out_v, [r_out + (41 + 2 * a)], c)

            move(in_v, r_in + 4, r_out + 46, 29)

            for j in range(8):
                ij = plsc.load_gather(in_v, [r_in + (33 + j)]).astype(jnp.int32) * TSTRIDE
                move(num_v, ij, r_out + (75 + 40 * j), 40)

    def out_window(ci):
        return out_hbm.at[pl.ds((tile_base + ci * CHUNK) * OUT_W, CHUNK * OUT_W)]

    compute(0, out_v0)
    pltpu.async_copy(out_v0, out_window(0), sem0)
    compute(1, out_v1)
    pltpu.async_copy(out_v1, out_window(1), sem1)

    @pl.loop(1, NPAIR)
    def _pair(p):
        ci0 = 2 * p
        pltpu.make_async_copy(out_v0, out_window(ci0 - 2), sem0).wait()
        compute(ci0, out_v0)
        pltpu.async_copy(out_v0, out_window(ci0), sem0)

        ci1 = 2 * p + 1
        pltpu.make_async_copy(out_v1, out_window(ci1 - 2), sem1).wait()
        compute(ci1, out_v1)
        pltpu.async_copy(out_v1, out_window(ci1), sem1)

    pltpu.make_async_copy(out_v0, out_window(NCHUNK - 2), sem0).wait()
    pltpu.make_async_copy(out_v1, out_window(NCHUNK - 1), sem1).wait()


def kernel(inputs, atom_table, num_table):
    flat_in = inputs.reshape(-1)
    mesh = plsc.VectorSubcoreMesh(core_axis_name="c", subcore_axis_name="s")
    fn = pl.kernel(
        _body,
        out_type=jax.ShapeDtypeStruct((ROWS * OUT_W,), jnp.float32),
        mesh=mesh,
        scratch_types=[
            pltpu.VMEM((100 * TSTRIDE,), jnp.float32),
            pltpu.VMEM((500 * TSTRIDE,), jnp.float32),
            pltpu.VMEM((CHUNK * IN_W,), jnp.float32),
            pltpu.VMEM((CHUNK * OUT_W,), jnp.float32),
            pltpu.VMEM((CHUNK * OUT_W,), jnp.float32),
            pltpu.SemaphoreType.DMA,
            pltpu.SemaphoreType.DMA,
        ],
        compiler_params=pltpu.CompilerParams(
            use_tc_tiling_on_sc=False, needs_layout_passes=False),
    )
    atom_p = jnp.pad(atom_table, ((0, 0), (0, TSTRIDE - 40))).reshape(-1)
    num_p = jnp.pad(num_table, ((0, 0), (0, TSTRIDE - 40))).reshape(-1)
    out = fn(flat_in, atom_p, num_p)
    return out.reshape(ROWS, 1, OUT_W)


# speedup vs baseline: 1.0346x; 1.0346x over previous
import math

import jax
import jax.numpy as jnp
from jax import lax
from jax.experimental import pallas as pl
from jax.experimental.pallas import tpu as pltpu
from jax.experimental.pallas import tpu_sc as plsc

B, L = 1024, 50
ROWS = B * L
IN_W = 41
OUT_W = 395
NC, NS, LANES = 2, 16, 16
NW = NC * NS
ROWS_PER_TILE = ROWS // NW
CHUNK = 80
NCHUNK = ROWS_PER_TILE // CHUNK
NPAIR = NCHUNK // 2
GROUPS = CHUNK // LANES

_TWO_PI = 2.0 * math.pi

TSTRIDE = 41


def _sincos_2pi(x):
    half = jnp.where(x >= 0.0, jnp.float32(0.5), jnp.float32(-0.5))
    k = (x + half).astype(jnp.int32)
    f = x - k.astype(jnp.float32)
    f4 = 4.0 * f
    half2 = jnp.where(f4 >= 0.0, jnp.float32(0.5), jnp.float32(-0.5))
    q = (f4 + half2).astype(jnp.int32)
    v = f - 0.25 * q.astype(jnp.float32)
    t = jnp.float32(_TWO_PI) * v
    t2 = t * t
    s = t * (1.0 + t2 * (-1.0 / 6.0 + t2 * (1.0 / 120.0 + t2 * (-1.0 / 5040.0))))
    c = 1.0 + t2 * (-0.5 + t2 * (1.0 / 24.0 + t2 * (-1.0 / 720.0 + t2 * (1.0 / 40320.0))))
    qm = jnp.bitwise_and(q, 3)
    sin_r = jnp.where(qm == 0, s, jnp.where(qm == 1, c, jnp.where(qm == 2, -s, -c)))
    cos_r = jnp.where(qm == 0, c, jnp.where(qm == 1, -s, jnp.where(qm == 2, -c, s)))
    return sin_r, cos_r


def _body(in_hbm, atom_hbm, num_hbm, out_hbm,
          atom_v, num_v, in_v0, in_v1, out_v0, out_v1, sem0, sem1, sem_in):
    wid = lax.axis_index("s") * NC + lax.axis_index("c")
    tile_base = wid * ROWS_PER_TILE

    pltpu.sync_copy(atom_hbm, atom_v)
    pltpu.sync_copy(num_hbm, num_v)

    def in_window(ci):
        return in_hbm.at[pl.ds((tile_base + ci * CHUNK) * IN_W, CHUNK * IN_W)]

    def compute(out_v, in_v):
        @pl.loop(0, GROUPS)
        def _group(g):
            rows = lax.iota(jnp.int32, LANES) + g * LANES
            r_in = rows * IN_W
            r_out = rows * OUT_W

            BATCH = 8

            def move(src, src_base, dst_base, width):
                for c0 in range(0, width, BATCH):
                    n = min(BATCH, width - c0)
                    vals = [plsc.load_gather(src, [src_base + (c0 + k)])
                            for k in range(n)]
                    for k in range(n):
                        plsc.store_scatter(out_v, [dst_base + (c0 + k)], vals[k])

            ia = plsc.load_gather(in_v, [r_in]).astype(jnp.int32) * TSTRIDE
            move(atom_v, ia, r_out, 40)

            for a in range(3):
                x = plsc.load_gather(in_v, [r_in + (1 + a)])
                s, c = _sincos_2pi(x)
                plsc.store_scatter(out_v, [r_out + (40 + 2 * a)], s)
                plsc.store_scatter(out_v, [r_out + (41 + 2 * a)], c)

            move(in_v, r_in + 4, r_out + 46, 29)

            for j in range(8):
                ij = plsc.load_gather(in_v, [r_in + (33 + j)]).astype(jnp.int32) * TSTRIDE
                move(num_v, ij, r_out + (75 + 40 * j), 40)

    def out_window(ci):
        return out_hbm.at[pl.ds((tile_base + ci * CHUNK) * OUT_W, CHUNK * OUT_W)]

    pltpu.sync_copy(in_window(0), in_v0)
    pltpu.async_copy(in_window(1), in_v1, sem_in)
    compute(out_v0, in_v0)
    pltpu.async_copy(out_v0, out_window(0), sem0)
    pltpu.make_async_copy(in_window(1), in_v1, sem_in).wait()
    compute(out_v1, in_v1)
    pltpu.async_copy(out_v1, out_window(1), sem1)

    @pl.loop(1, NPAIR)
    def _pair(p):
        ci0 = 2 * p
        ci1 = 2 * p + 1
        pltpu.sync_copy(in_window(ci0), in_v0)
        pltpu.async_copy(in_window(ci1), in_v1, sem_in)
        pltpu.make_async_copy(out_v0, out_window(ci0 - 2), sem0).wait()
        compute(out_v0, in_v0)
        pltpu.async_copy(out_v0, out_window(ci0), sem0)

        pltpu.make_async_copy(out_v1, out_window(ci1 - 2), sem1).wait()
        pltpu.make_async_copy(in_window(ci1), in_v1, sem_in).wait()
        compute(out_v1, in_v1)
        pltpu.async_copy(out_v1, out_window(ci1), sem1)

    pltpu.make_async_copy(out_v0, out_window(NCHUNK - 2), sem0).wait()
    pltpu.make_async_copy(out_v1, out_window(NCHUNK - 1), sem1).wait()


def kernel(inputs, atom_table, num_table):
    flat_in = inputs.reshape(-1)
    mesh = plsc.VectorSubcoreMesh(core_axis_name="c", subcore_axis_name="s")
    fn = pl.kernel(
        _body,
        out_type=jax.ShapeDtypeStruct((ROWS * OUT_W,), jnp.float32),
        mesh=mesh,
        scratch_types=[
            pltpu.VMEM((100 * TSTRIDE,), jnp.float32),
            pltpu.VMEM((500 * TSTRIDE,), jnp.float32),
            pltpu.VMEM((CHUNK * IN_W,), jnp.float32),
            pltpu.VMEM((CHUNK * IN_W,), jnp.float32),
            pltpu.VMEM((CHUNK * OUT_W,), jnp.float32),
            pltpu.VMEM((CHUNK * OUT_W,), jnp.float32),
            pltpu.SemaphoreType.DMA,
            pltpu.SemaphoreType.DMA,
            pltpu.SemaphoreType.DMA,
        ],
        compiler_params=pltpu.CompilerParams(
            use_tc_tiling_on_sc=False, needs_layout_passes=False),
    )
    atom_p = jnp.pad(atom_table, ((0, 0), (0, TSTRIDE - 40))).reshape(-1)
    num_p = jnp.pad(num_table, ((0, 0), (0, TSTRIDE - 40))).reshape(-1)
    out = fn(flat_in, atom_p, num_p)
    return out.reshape(ROWS, 1, OUT_W)
